# baseline (speedup 1.0000x reference)
import functools

import jax
import jax.numpy as jnp
from jax import lax
from jax.experimental import pallas as pl
from jax.experimental.pallas import tpu as pltpu
from jax.experimental.pallas import tpu_sc as plsc

EMB = 1024
LANES = 16
VPR = EMB // LANES

_info = plsc.get_sparse_core_info()
NC, NS = _info.num_cores, _info.num_subcores
NW = NC * NS

C = 16
CH = C // 2


def _make_kernel(n_rows: int):
    rows_per_w = n_rows // NW
    n_chunks = rows_per_w // C
    assert n_chunks % 2 == 0 and n_chunks >= 8
    mesh = plsc.VectorSubcoreMesh(core_axis_name="c", subcore_axis_name="s")

    buf = lambda: pltpu.VMEM((C, EMB), jnp.float32)
    sem = pltpu.SemaphoreType.DMA

    @functools.partial(
        pl.kernel,
        mesh=mesh,
        out_type=jax.ShapeDtypeStruct((n_rows, EMB), jnp.float32),
        scratch_types=[
            pltpu.VMEM((rows_per_w,), jnp.int32),
            [buf() for _ in range(2)],
            [buf() for _ in range(2)],
            buf(),
            pltpu.VMEM_SHARED((NS, 2, C, EMB), jnp.float32),
            [sem for _ in range(2)],
            [sem for _ in range(2)],
            [sem for _ in range(2)],
            [sem for _ in range(2)],
        ],
    )
    def k(emb_hbm, ts_hbm, table_hbm, out_hbm, idx_v,
          embs, rows, outv, spe, sgs, sps, sxs, sos):
        cid = lax.axis_index("c")
        sid = lax.axis_index("s")
        wid = sid * NC + cid
        base = wid * rows_per_w
        pltpu.sync_copy(ts_hbm.at[pl.ds(base, rows_per_w)], idx_v)

        def start_g(ci, b):
            pltpu.async_copy(
                table_hbm.at[idx_v.at[pl.ds(ci * C, C)]], rows[b], sgs[b])

        def wait_g(b):
            pltpu.make_async_copy(
                table_hbm.at[idx_v.at[pl.ds(0, C)]], rows[b], sgs[b]).wait()

        def start_sp(ci, b):
            pltpu.async_copy(
                emb_hbm.at[pl.ds(base + ci * C, C)], spe.at[sid, b], sps[b])

        def wait_sp(b):
            pltpu.make_async_copy(
                emb_hbm.at[pl.ds(base, C)], spe.at[sid, b], sps[b]).wait()

        def start_x(b):
            pltpu.async_copy(spe.at[sid, b], embs[b], sxs[b])

        def wait_x(b):
            pltpu.make_async_copy(spe.at[sid, b], embs[b], sxs[b]).wait()

        def add_half(b, h):
            @pl.loop(0, CH)
            def _(r):
                for j in range(VPR):
                    sl = pl.ds(j * LANES, LANES)
                    outv[h * CH + r, sl] = rows[b][h * CH + r, sl] + embs[b][h * CH + r, sl]

        def start_oh(ci, h):
            pltpu.async_copy(
                outv.at[pl.ds(h * CH, CH)],
                out_hbm.at[pl.ds(base + ci * C + h * CH, CH)], sos[h])

        def wait_oh(h):
            pltpu.make_async_copy(
                outv.at[pl.ds(0, CH)], out_hbm.at[pl.ds(base, CH)], sos[h]).wait()

        def step(c, b, prefetch, chain_x, drain_out):
            wait_x(b)
            wait_g(b)
            if prefetch:
                start_sp(c + 2, b)
            if chain_x:
                wait_sp(1 - b)
                start_x(1 - b)
            for h in range(2):
                if drain_out:
                    wait_oh(h)
                add_half(b, h)
                start_oh(c, h)
            if prefetch:
                start_g(c + 2, b)

        start_sp(0, 0)
        start_sp(1, 1)
        start_g(0, 0)
        start_g(1, 1)
        wait_sp(0)
        start_x(0)

        step(0, 0, True, True, False)
        step(1, 1, True, True, True)

        @pl.loop(2, n_chunks - 2, step=2)
        def body(ci):
            for b in (0, 1):
                step(ci + b, b, True, True, True)

        step(n_chunks - 2, 0, False, True, True)
        step(n_chunks - 1, 1, False, False, True)
        wait_oh(0)
        wait_oh(1)

    return k


@jax.jit
def kernel(emb_vec, timesteps, pos_table):
    b, s, e = emb_vec.shape
    n = b * s
    emb2 = emb_vec.reshape(n, e)
    ts1 = timesteps.reshape(n)
    out = _make_kernel(n)(emb2, ts1, pos_table)
    return out.reshape(b, s, e)


# speedup vs baseline: 1.0344x; 1.0344x over previous
import functools

import jax
import jax.numpy as jnp
from jax import lax
from jax.experimental import pallas as pl
from jax.experimental.pallas import tpu as pltpu
from jax.experimental.pallas import tpu_sc as plsc

EMB = 1024
LANES = 16
VPR = EMB // LANES

_info = plsc.get_sparse_core_info()
NC, NS = _info.num_cores, _info.num_subcores
NW = NC * NS


def _make_kernel(n_rows: int, max_len: int, c_rows: int):
    rows_per_w = n_rows // NW
    n_chunks = rows_per_w // c_rows
    assert n_chunks % 2 == 0 and n_chunks >= 4
    mesh = plsc.VectorSubcoreMesh(core_axis_name="c", subcore_axis_name="s")

    buf = lambda: pltpu.VMEM((c_rows, EMB), jnp.float32)

    @functools.partial(
        pl.kernel,
        mesh=mesh,
        out_type=jax.ShapeDtypeStruct((n_rows, EMB), jnp.float32),
        scratch_types=[
            pltpu.VMEM((rows_per_w,), jnp.int32),
            buf(), buf(),
            buf(), buf(),
            buf(), buf(),
            pltpu.SemaphoreType.DMA, pltpu.SemaphoreType.DMA,
            pltpu.SemaphoreType.DMA, pltpu.SemaphoreType.DMA,
            pltpu.SemaphoreType.DMA, pltpu.SemaphoreType.DMA,
        ],
    )
    def k(emb_hbm, ts_hbm, table_hbm, out_hbm, idx_v,
          e0, e1, r0, r1, o0, o1, se0, se1, sg0, sg1, so0, so1):
        wid = lax.axis_index("s") * NC + lax.axis_index("c")
        base = wid * rows_per_w
        pltpu.sync_copy(ts_hbm.at[pl.ds(base, rows_per_w)], idx_v)

        embs, rows, outs = (e0, e1), (r0, r1), (o0, o1)
        ses, sgs, sos = (se0, se1), (sg0, sg1), (so0, so1)

        def start_in(ci, b):
            pltpu.async_copy(
                table_hbm.at[idx_v.at[pl.ds(ci * c_rows, c_rows)]], rows[b], sgs[b])
            pltpu.async_copy(
                emb_hbm.at[pl.ds(base + ci * c_rows, c_rows)], embs[b], ses[b])

        def wait_in(b):
            pltpu.make_async_copy(
                table_hbm.at[idx_v.at[pl.ds(0, c_rows)]], rows[b], sgs[b]).wait()
            pltpu.make_async_copy(
                emb_hbm.at[pl.ds(base, c_rows)], embs[b], ses[b]).wait()

        def add(b):
            @pl.loop(0, c_rows)
            def _(r):
                for j in range(VPR):
                    sl = pl.ds(j * LANES, LANES)
                    outs[b][r, sl] = rows[b][r, sl] + embs[b][r, sl]

        def start_out(ci, b):
            pltpu.async_copy(outs[b], out_hbm.at[pl.ds(base + ci * c_rows, c_rows)], sos[b])

        def wait_out(b):
            pltpu.make_async_copy(outs[b], out_hbm.at[pl.ds(base, c_rows)], sos[b]).wait()

        start_in(0, 0)
        start_in(1, 1)
        for b in (0, 1):
            wait_in(b)
            add(b)
            start_in(b + 2, b)
            start_out(b, b)

        @pl.loop(2, n_chunks - 2, step=2)
        def body(ci):
            for b in (0, 1):
                cur = ci + b
                wait_in(b)
                wait_out(b)
                add(b)
                start_in(cur + 2, b)
                start_out(cur, b)

        for b in (0, 1):
            wait_in(b)
            wait_out(b)
            add(b)
            start_out(n_chunks - 2 + b, b)
        wait_out(0)
        wait_out(1)

    return k


@jax.jit
def kernel(emb_vec, timesteps, pos_table):
    b, s, e = emb_vec.shape
    n = b * s
    emb2 = emb_vec.reshape(n, e)
    ts1 = timesteps.reshape(n)
    out = _make_kernel(n, pos_table.shape[0], 16)(emb2, ts1, pos_table)
    return out.reshape(b, s, e)
